# baseline (speedup 1.0000x reference)
import functools
import jax
import jax.numpy as jnp
from jax import lax
from jax.experimental import pallas as pl
from jax.experimental.pallas import tpu as pltpu, tpu_sc as plsc

EMB_DIM = 64
SCALE = 8.0

NC = 2
NS = 16
NW = NC * NS
CH = 128


@functools.partial(jax.jit, static_argnames=("nch",))
def _emb_lookup(table, idx, nch):
    @functools.partial(
        pl.kernel,
        out_type=jax.ShapeDtypeStruct((NW, nch, CH, EMB_DIM), jnp.float32),
        mesh=plsc.VectorSubcoreMesh(
            core_axis_name="c", subcore_axis_name="s",
            num_cores=NC, num_subcores=NS,
        ),
        scratch_types=[
            pltpu.VMEM((nch, CH), jnp.int32),
            pltpu.VMEM((CH, EMB_DIM), jnp.float32),
            pltpu.SemaphoreType.DMA,
        ],
        compiler_params=pltpu.CompilerParams(use_tc_tiling_on_sc=False),
    )
    def body(table_hbm, idx_hbm, out_hbm, idx_v, row_v, sem):
        wid = lax.axis_index("s") * NC + lax.axis_index("c")
        pltpu.sync_copy(idx_hbm.at[wid], idx_v)

        @pl.loop(0, nch)
        def _chunk(j):
            pltpu.async_copy(table_hbm.at[idx_v.at[j]], row_v, sem).wait()

            @pl.loop(0, CH)
            def _row(r):
                for cc in range(EMB_DIM // 16):
                    sl = pl.ds(cc * 16, 16)
                    row_v[r, sl] = row_v[r, sl] * SCALE

            pltpu.sync_copy(row_v, out_hbm.at[wid, j])

    return body(table, idx)


def kernel(x, table):
    b, s = x.shape
    total = b * s
    assert total % (NW * CH) == 0
    nch = total // (NW * CH)
    xf = x.reshape(NW, nch, CH).astype(jnp.int32)
    out = _emb_lookup(table, xf, nch)
    return out.reshape(b, s, EMB_DIM)


# speedup vs baseline: 1.1904x; 1.1904x over previous
import functools
import jax
import jax.numpy as jnp
from jax import lax
from jax.experimental import pallas as pl
from jax.experimental.pallas import tpu as pltpu, tpu_sc as plsc

EMB_DIM = 64
SCALE = 8.0

NC = 2
NS = 16
NW = NC * NS
CH = 128
NBUF = 2


@functools.partial(jax.jit, static_argnames=("nch",))
def _emb_lookup(table, idx, nch):
    @functools.partial(
        pl.kernel,
        out_type=jax.ShapeDtypeStruct((NW, nch, CH, EMB_DIM), jnp.float32),
        mesh=plsc.VectorSubcoreMesh(
            core_axis_name="c", subcore_axis_name="s",
            num_cores=NC, num_subcores=NS,
        ),
        scratch_types=[
            pltpu.VMEM((nch, CH), jnp.int32),
            pltpu.VMEM((NBUF, CH, EMB_DIM), jnp.float32),
            pltpu.VMEM((NBUF, CH, EMB_DIM), jnp.float32),
            pltpu.SemaphoreType.DMA,
            pltpu.SemaphoreType.DMA,
            pltpu.SemaphoreType.DMA,
            pltpu.SemaphoreType.DMA,
        ],
        compiler_params=pltpu.CompilerParams(use_tc_tiling_on_sc=False),
    )
    def body(table_hbm, idx_hbm, out_hbm, idx_v, in_v, out_v,
             g0, g1, s0, s1):
        gsem = (g0, g1)
        ssem = (s0, s1)
        wid = lax.axis_index("s") * NC + lax.axis_index("c")
        pltpu.sync_copy(idx_hbm.at[wid], idx_v)

        def start_gather(g, b):
            pltpu.async_copy(table_hbm.at[idx_v.at[g]], in_v.at[b], gsem[b])

        def process(g, b, fetch, first):
            pltpu.make_async_copy(
                table_hbm.at[idx_v.at[g]], in_v.at[b], gsem[b]).wait()
            if not first:
                pltpu.make_async_copy(
                    out_v.at[b], out_hbm.at[wid, g], ssem[b]).wait()

            @plsc.parallel_loop(0, CH, unroll=4)
            def _row(r):
                for cc in range(EMB_DIM // 16):
                    sl = pl.ds(cc * 16, 16)
                    out_v[b, r, sl] = in_v[b, r, sl] * SCALE

            pltpu.async_copy(out_v.at[b], out_hbm.at[wid, g], ssem[b])
            if fetch:
                start_gather(g + NBUF, b)

        for b in range(NBUF):
            start_gather(b, b)
        for b in range(NBUF):
            process(b, b, fetch=True, first=True)

        @pl.loop(0, (nch - 2 * NBUF) // NBUF)
        def _main(t):
            g0i = NBUF + NBUF * t
            for db in range(NBUF):
                process(g0i + db, db, fetch=True, first=False)

        for db in range(NBUF):
            g = nch - NBUF + db
            process(g, db, fetch=False, first=False)
        for b in range(NBUF):
            pltpu.make_async_copy(
                out_v.at[b], out_hbm.at[wid, nch - NBUF + b], ssem[b]).wait()

    return body(table, idx)


def kernel(x, table):
    b, s = x.shape
    total = b * s
    assert total % (NW * CH) == 0
    nch = total // (NW * CH)
    xf = x.reshape(NW, nch, CH).astype(jnp.int32)
    out = _emb_lookup(table, xf, nch)
    return out.reshape(b, s, EMB_DIM)


# speedup vs baseline: 1.2090x; 1.0156x over previous
import functools
import jax
import jax.numpy as jnp
from jax import lax
from jax.experimental import pallas as pl
from jax.experimental.pallas import tpu as pltpu, tpu_sc as plsc

EMB_DIM = 64
SCALE = 8.0

NC = 2
NS = 16
NW = NC * NS
CH = 128
NBUF = 4


@functools.partial(jax.jit, static_argnames=("nch",))
def _emb_lookup(table, idx, nch):
    @functools.partial(
        pl.kernel,
        out_type=jax.ShapeDtypeStruct((NW, nch, CH, EMB_DIM), jnp.float32),
        mesh=plsc.VectorSubcoreMesh(
            core_axis_name="c", subcore_axis_name="s",
            num_cores=NC, num_subcores=NS,
        ),
        scratch_types=(
            [pltpu.VMEM((nch, CH), jnp.int32)]
            + [pltpu.VMEM((NBUF, CH, EMB_DIM), jnp.float32)] * 2
            + [pltpu.SemaphoreType.DMA] * (2 * NBUF)
        ),
        compiler_params=pltpu.CompilerParams(use_tc_tiling_on_sc=False),
    )
    def body(table_hbm, idx_hbm, out_hbm, idx_v, in_v, out_v, *sems):
        gsem = sems[:NBUF]
        ssem = sems[NBUF:]
        wid = lax.axis_index("s") * NC + lax.axis_index("c")
        pltpu.sync_copy(idx_hbm.at[wid], idx_v)

        def start_gather(g, b):
            pltpu.async_copy(table_hbm.at[idx_v.at[g]], in_v.at[b], gsem[b])

        def process(g, b, fetch, first):
            pltpu.make_async_copy(
                table_hbm.at[idx_v.at[g]], in_v.at[b], gsem[b]).wait()
            if not first:
                pltpu.make_async_copy(
                    out_v.at[b], out_hbm.at[wid, g], ssem[b]).wait()

            @plsc.parallel_loop(0, CH, unroll=4)
            def _row(r):
                for cc in range(EMB_DIM // 16):
                    sl = pl.ds(cc * 16, 16)
                    out_v[b, r, sl] = in_v[b, r, sl] * SCALE

            pltpu.async_copy(out_v.at[b], out_hbm.at[wid, g], ssem[b])
            if fetch:
                start_gather(g + NBUF, b)

        for b in range(NBUF):
            start_gather(b, b)
        for b in range(NBUF):
            process(b, b, fetch=True, first=True)

        @pl.loop(0, (nch - 2 * NBUF) // NBUF)
        def _main(t):
            g0i = NBUF + NBUF * t
            for db in range(NBUF):
                process(g0i + db, db, fetch=True, first=False)

        for db in range(NBUF):
            g = nch - NBUF + db
            process(g, db, fetch=False, first=False)
        for b in range(NBUF):
            pltpu.make_async_copy(
                out_v.at[b], out_hbm.at[wid, nch - NBUF + b], ssem[b]).wait()

    return body(table, idx)


def kernel(x, table):
    b, s = x.shape
    total = b * s
    assert total % (NW * CH) == 0
    nch = total // (NW * CH)
    xf = x.reshape(NW, nch, CH).astype(jnp.int32)
    out = _emb_lookup(table, xf, nch)
    return out.reshape(b, s, EMB_DIM)
